# baseline (device time: 69583 ns/iter reference)
import jax
import jax.numpy as jnp
from jax import lax
from jax.experimental import pallas as pl
from jax.experimental.pallas import tpu as pltpu

N_DEV = 4
SQ = 1024
SKV = 1024
HQ = 8
DH = 128
D_MODEL = 1024
SCALE = 0.08838834764831843
BLOCK = 64
NSTRIDE = 4
NREP = 4
GROUP = NREP * BLOCK
NCHUNK = 8
CHUNK = SQ // NCHUNK


def _body(x_ref, wq_ref, k_ref, v_ref, wo_ref, out_ref,
          q_ref, wob_ref, ctx_ref, chnk_ref,
          rs_send, rs_recv, ag_send, ag_recv,
          rs_send_sems, rs_recv_sems, ag_send_sems, ag_recv_sems):
    my = lax.axis_index("i")

    barrier_sem = pltpu.get_barrier_semaphore()
    for d in range(1, N_DEV):
        pl.semaphore_signal(barrier_sem, inc=1,
                            device_id=((my + d) % N_DEV,),
                            device_id_type=pl.DeviceIdType.MESH)
    pl.semaphore_wait(barrier_sem, N_DEV - 1)

    q_ref[...] = jnp.dot(x_ref[...].astype(jnp.bfloat16),
                         wq_ref[...].astype(jnp.bfloat16),
                         preferred_element_type=jnp.float32
                         ).astype(jnp.bfloat16)
    wob_ref[...] = wo_ref[...].astype(jnp.bfloat16)

    for h in range(HQ):
        for s in range(NSTRIDE):
            rows = slice(s * GROUP, (s + 1) * GROUP)
            qs = q_ref[rows, h * DH:(h + 1) * DH]
            kb = k_ref[h, s].astype(jnp.bfloat16)
            vb = v_ref[h, s].astype(jnp.bfloat16)
            sc = lax.dot_general(qs, kb, (((1,), (1,)), ((), ())),
                                 preferred_element_type=jnp.float32)
            w = jnp.exp(sc * SCALE)
            rsum = 1.0 / jnp.sum(w, axis=-1, keepdims=True)
            ctx = jnp.dot(w.astype(jnp.bfloat16), vb,
                          preferred_element_type=jnp.float32)
            ctx_ref[rows, h * DH:(h + 1) * DH] = (
                (ctx * rsum).astype(jnp.bfloat16))

    for p in range(NCHUNK):
        rnd = p // 4
        t = (my + 1 + p) % 4 + 4 * rnd

        s0 = (2 * t) % 4
        r64 = (t // 2) * 64
        chnk_ref[0:BLOCK, :] = ctx_ref[pl.ds(s0 * GROUP + r64, BLOCK), :]
        chnk_ref[BLOCK:2 * BLOCK, :] = (
            ctx_ref[pl.ds((s0 + 1) * GROUP + r64, BLOCK), :])

        partial = jnp.dot(chnk_ref[...], wob_ref[...],
                          preferred_element_type=jnp.float32)

        if p % 4 != 3:
            rs_send[p] = partial.astype(jnp.bfloat16)
            pltpu.make_async_remote_copy(
                src_ref=rs_send.at[p], dst_ref=rs_recv.at[my + 4 * rnd],
                send_sem=rs_send_sems.at[p],
                recv_sem=rs_recv_sems.at[my + 4 * rnd],
                device_id=(t % 4,), device_id_type=pl.DeviceIdType.MESH,
            ).start()
        else:
            for s_ in range(N_DEV):
                @pl.when(s_ != my)
                def _():
                    pltpu.make_async_remote_copy(
                        src_ref=rs_send.at[p],
                        dst_ref=rs_recv.at[s_ + 4 * rnd],
                        send_sem=rs_send_sems.at[p],
                        recv_sem=rs_recv_sems.at[s_ + 4 * rnd],
                        device_id=(s_,),
                        device_id_type=pl.DeviceIdType.MESH,
                    ).wait_recv()
            own = partial
            for s_ in range(N_DEV):
                own = own + jnp.where(
                    s_ == my, jnp.float32(0.0),
                    rs_recv[s_ + 4 * rnd].astype(jnp.float32))
            out_ref[pl.ds(t * CHUNK, CHUNK), :] = own
            ag_send[rnd] = own.astype(jnp.bfloat16)
            for o in range(N_DEV):
                @pl.when(o != my)
                def _():
                    pltpu.make_async_remote_copy(
                        src_ref=ag_send.at[rnd],
                        dst_ref=ag_recv.at[my + 4 * rnd],
                        send_sem=ag_send_sems.at[o + 4 * rnd],
                        recv_sem=ag_recv_sems.at[my + 4 * rnd],
                        device_id=(o,),
                        device_id_type=pl.DeviceIdType.MESH,
                    ).start()

    for p in range(NCHUNK):
        if p % 4 != 3:
            rnd = p // 4
            pltpu.make_async_remote_copy(
                src_ref=rs_send.at[p], dst_ref=rs_recv.at[my + 4 * rnd],
                send_sem=rs_send_sems.at[p],
                recv_sem=rs_recv_sems.at[my + 4 * rnd],
                device_id=((my + 1 + p) % 4,),
                device_id_type=pl.DeviceIdType.MESH,
            ).wait_send()

    for rnd in range(2):
        for o in range(N_DEV):
            @pl.when(o != my)
            def _():
                pltpu.make_async_remote_copy(
                    src_ref=ag_send.at[rnd], dst_ref=ag_recv.at[o + 4 * rnd],
                    send_sem=ag_send_sems.at[o + 4 * rnd],
                    recv_sem=ag_recv_sems.at[o + 4 * rnd],
                    device_id=(o,), device_id_type=pl.DeviceIdType.MESH,
                ).wait_recv()
                out_ref[(o + 4 * rnd) * CHUNK:(o + 4 * rnd + 1) * CHUNK,
                        :] = ag_recv[o + 4 * rnd].astype(jnp.float32)

    for rnd in range(2):
        for o in range(N_DEV):
            @pl.when(o != my)
            def _():
                pltpu.make_async_remote_copy(
                    src_ref=ag_send.at[rnd], dst_ref=ag_recv.at[my + 4 * rnd],
                    send_sem=ag_send_sems.at[o + 4 * rnd],
                    recv_sem=ag_recv_sems.at[my + 4 * rnd],
                    device_id=(o,), device_id_type=pl.DeviceIdType.MESH,
                ).wait_send()


def kernel(x, Wq, K_ext, V_ext, Wo):
    my = lax.axis_index("i")
    xp = x[0].reshape(NREP, NSTRIDE, BLOCK, D_MODEL)
    xp = jnp.transpose(xp, (1, 0, 2, 3)).reshape(SQ, D_MODEL)

    def group_kv(t):
        g = lax.dynamic_slice_in_dim(t[0], my * HQ, HQ, axis=1)
        g = g.reshape(NREP, NSTRIDE, BLOCK, HQ, DH)
        return jnp.transpose(g, (3, 1, 0, 2, 4)).reshape(
            HQ, NSTRIDE, GROUP, DH)

    k = group_kv(K_ext)
    v = group_kv(V_ext)

    out = pl.pallas_call(
        _body,
        out_shape=jax.ShapeDtypeStruct((SQ, D_MODEL), jnp.float32),
        in_specs=[pl.BlockSpec(memory_space=pltpu.VMEM)] * 5,
        out_specs=pl.BlockSpec(memory_space=pltpu.VMEM),
        scratch_shapes=[
            pltpu.VMEM((SQ, HQ * DH), jnp.bfloat16),
            pltpu.VMEM((HQ * DH, D_MODEL), jnp.bfloat16),
            pltpu.VMEM((SQ, HQ * DH), jnp.bfloat16),
            pltpu.VMEM((CHUNK, HQ * DH), jnp.bfloat16),
            pltpu.VMEM((NCHUNK, CHUNK, D_MODEL), jnp.bfloat16),
            pltpu.VMEM((NCHUNK, CHUNK, D_MODEL), jnp.bfloat16),
            pltpu.VMEM((2, CHUNK, D_MODEL), jnp.bfloat16),
            pltpu.VMEM((NCHUNK, CHUNK, D_MODEL), jnp.bfloat16),
            pltpu.SemaphoreType.DMA((NCHUNK,)),
            pltpu.SemaphoreType.DMA((NCHUNK,)),
            pltpu.SemaphoreType.DMA((NCHUNK,)),
            pltpu.SemaphoreType.DMA((NCHUNK,)),
        ],
        compiler_params=pltpu.CompilerParams(
            collective_id=0,
            vmem_limit_bytes=120 * 1024 * 1024,
        ),
    )(xp, Wq, k, v, Wo)
    return out[None]
